# Initial kernel scaffold; baseline (speedup 1.0000x reference)
import jax
import jax.numpy as jnp
from jax.experimental import pallas as pl


def kernel(pos_indices, emb_table, W, b):
    raise NotImplementedError("write your pallas kernel here")


# speedup vs baseline: 73.0265x; 73.0265x over previous
import functools

import jax
import jax.numpy as jnp
from jax import lax
from jax.experimental import pallas as pl
from jax.experimental.pallas import tpu as pltpu
from jax.experimental.pallas import tpu_sc as plsc

_NC = 2
_NS = 16
_NW = _NC * _NS
_LANES = 16
_CHUNK = 4096


def _prob_body(tab_ref, w_ref, b_ref, out_ref):
    t = tab_ref[...]
    w = w_ref[...]
    logits = jnp.sum(t * w, axis=1, keepdims=True) + b_ref[...]
    out_ref[...] = jax.nn.sigmoid(logits)


def _compute_prob_table(emb_table, W, b):
    rows = emb_table.shape[0]
    return pl.pallas_call(
        _prob_body,
        out_shape=jax.ShapeDtypeStruct((rows, 1), jnp.float32),
    )(emb_table, W, b.reshape(1, 1))


@functools.lru_cache(maxsize=None)
def _make_sc_gather(n, table_rows):
    assert n % (_NW * _CHUNK) == 0
    per_w = n // _NW
    nchunk = per_w // _CHUNK
    mesh = plsc.VectorSubcoreMesh(core_axis_name="c", subcore_axis_name="s")

    @functools.partial(
        pl.kernel,
        out_type=jax.ShapeDtypeStruct((n,), jnp.float32),
        mesh=mesh,
        scratch_types=[
            pltpu.VMEM((table_rows,), jnp.float32),
            pltpu.VMEM((_CHUNK,), jnp.int32),
            pltpu.VMEM((_CHUNK,), jnp.float32),
        ],
        compiler_params=pltpu.CompilerParams(needs_layout_passes=False),
    )
    def gather_kernel(ptab_hbm, idx_hbm, out_hbm, ptab_v, idx_v, out_v):
        wid = lax.axis_index("s") * _NC + lax.axis_index("c")
        base = wid * per_w
        pltpu.sync_copy(ptab_hbm, ptab_v)

        def chunk_body(g, carry):
            off = base + g * _CHUNK
            pltpu.sync_copy(idx_hbm.at[pl.ds(off, _CHUNK)], idx_v)

            def inner(i, c):
                sl = pl.ds(i * _LANES, _LANES)
                out_v[sl] = plsc.load_gather(ptab_v, [idx_v[sl]])
                return c

            lax.fori_loop(0, _CHUNK // _LANES, inner, 0, unroll=8)
            pltpu.sync_copy(out_v, out_hbm.at[pl.ds(off, _CHUNK)])
            return carry

        lax.fori_loop(0, nchunk, chunk_body, 0)

    return gather_kernel


def kernel(pos_indices, emb_table, W, b):
    batch, seqlen = pos_indices.shape
    n = batch * seqlen
    idx = pos_indices.reshape(n).astype(jnp.int32)
    ptab = _compute_prob_table(emb_table, W, b).reshape(-1)
    out = _make_sc_gather(n, emb_table.shape[0])(ptab, idx)
    return out.reshape(batch, seqlen, 1)


# speedup vs baseline: 84.0013x; 1.1503x over previous
import functools

import jax
import jax.numpy as jnp
from jax import lax
from jax.experimental import pallas as pl
from jax.experimental.pallas import tpu as pltpu
from jax.experimental.pallas import tpu_sc as plsc

_NC = 2
_NS = 16
_NW = _NC * _NS
_LANES = 16
_CHUNK = 12800


def _prob_body(tab_ref, w_ref, b_ref, out_ref):
    t = tab_ref[...]
    w = w_ref[...]
    logits = jnp.sum(t * w, axis=1, keepdims=True) + b_ref[...]
    out_ref[...] = jax.nn.sigmoid(logits)


def _compute_prob_table(emb_table, W, b):
    rows = emb_table.shape[0]
    return pl.pallas_call(
        _prob_body,
        out_shape=jax.ShapeDtypeStruct((rows, 1), jnp.float32),
    )(emb_table, W, b.reshape(1, 1))


@functools.lru_cache(maxsize=None)
def _make_sc_gather(n, table_rows):
    assert n % (_NW * _CHUNK) == 0
    per_w = n // _NW
    nchunk = per_w // _CHUNK
    mesh = plsc.VectorSubcoreMesh(core_axis_name="c", subcore_axis_name="s")

    @functools.partial(
        pl.kernel,
        out_type=jax.ShapeDtypeStruct((n,), jnp.float32),
        mesh=mesh,
        scratch_types=[
            pltpu.VMEM((table_rows,), jnp.float32),
            pltpu.VMEM((_CHUNK,), jnp.int32),
            pltpu.VMEM((_CHUNK,), jnp.int32),
            pltpu.VMEM((_CHUNK,), jnp.float32),
            pltpu.VMEM((_CHUNK,), jnp.float32),
            pltpu.SemaphoreType.DMA,
            pltpu.SemaphoreType.DMA,
            pltpu.SemaphoreType.DMA,
            pltpu.SemaphoreType.DMA,
        ],
        compiler_params=pltpu.CompilerParams(needs_layout_passes=False),
    )
    def gather_kernel(ptab_hbm, idx_hbm, out_hbm, ptab_v,
                      idx_v0, idx_v1, out_v0, out_v1,
                      sem_i0, sem_i1, sem_o0, sem_o1):
        wid = lax.axis_index("s") * _NC + lax.axis_index("c")
        base = wid * per_w
        idx_bufs, out_bufs = (idx_v0, idx_v1), (out_v0, out_v1)
        sem_in, sem_out = (sem_i0, sem_i1), (sem_o0, sem_o1)
        pltpu.sync_copy(ptab_hbm, ptab_v)

        def in_copy(g):
            return pltpu.async_copy(
                idx_hbm.at[pl.ds(base + g * _CHUNK, _CHUNK)],
                idx_bufs[g % 2], sem_in[g % 2])

        in_handles = [in_copy(0)]
        out_handles = [None] * nchunk
        for g in range(nchunk):
            cur = g % 2
            if g + 1 < nchunk:
                in_handles.append(in_copy(g + 1))
            in_handles[g].wait()
            if g >= 2:
                out_handles[g - 2].wait()
            idx_v, out_v = idx_bufs[cur], out_bufs[cur]

            def inner(i, c, idx_v=idx_v, out_v=out_v):
                sl = pl.ds(i * _LANES, _LANES)
                out_v[sl] = plsc.load_gather(ptab_v, [idx_v[sl]])
                return c

            lax.fori_loop(0, _CHUNK // _LANES, inner, 0, unroll=8)
            out_handles[g] = pltpu.async_copy(
                out_v, out_hbm.at[pl.ds(base + g * _CHUNK, _CHUNK)],
                sem_out[cur])
        for g in range(max(0, nchunk - 2), nchunk):
            out_handles[g].wait()

    return gather_kernel


def kernel(pos_indices, emb_table, W, b):
    batch, seqlen = pos_indices.shape
    n = batch * seqlen
    idx = pos_indices.reshape(n).astype(jnp.int32)
    ptab = _compute_prob_table(emb_table, W, b).reshape(-1)
    out = _make_sc_gather(n, emb_table.shape[0])(ptab, idx)
    return out.reshape(batch, seqlen, 1)


# speedup vs baseline: 153.4406x; 1.8266x over previous
import functools

import jax
import jax.numpy as jnp
from jax import lax
from jax.experimental import pallas as pl
from jax.experimental.pallas import tpu as pltpu
from jax.experimental.pallas import tpu_sc as plsc

_NC = 2
_NS = 16
_NW = _NC * _NS
_LANES = 16
_RCHUNK = 64


def _prob_body(tab_ref, w_ref, b_ref, out_ref):
    t = tab_ref[...]
    w = w_ref[...]
    logits = jnp.sum(t * w, axis=1, keepdims=True) + b_ref[...]
    out_ref[...] = jax.nn.sigmoid(logits)


def _compute_prob_table(emb_table, W, b):
    rows = emb_table.shape[0]
    return pl.pallas_call(
        _prob_body,
        out_shape=jax.ShapeDtypeStruct((rows, 1), jnp.float32),
    )(emb_table, W, b.reshape(1, 1))


@functools.lru_cache(maxsize=None)
def _make_sc_gather(batch, seqlen, table_rows):
    assert batch % (_NW * _RCHUNK) == 0
    rows_per_w = batch // _NW
    nchunk = rows_per_w // _RCHUNK
    starts = list(range(0, (seqlen // _LANES) * _LANES, _LANES))
    if seqlen % _LANES:
        starts.append(seqlen - _LANES)
    mesh = plsc.VectorSubcoreMesh(core_axis_name="c", subcore_axis_name="s")

    @functools.partial(
        pl.kernel,
        out_type=jax.ShapeDtypeStruct((batch, seqlen), jnp.float32),
        mesh=mesh,
        scratch_types=[
            pltpu.VMEM((table_rows,), jnp.float32),
            pltpu.VMEM((_RCHUNK, seqlen), jnp.int32),
            pltpu.VMEM((_RCHUNK, seqlen), jnp.int32),
            pltpu.VMEM((_RCHUNK, seqlen), jnp.float32),
            pltpu.VMEM((_RCHUNK, seqlen), jnp.float32),
            pltpu.SemaphoreType.DMA,
            pltpu.SemaphoreType.DMA,
            pltpu.SemaphoreType.DMA,
            pltpu.SemaphoreType.DMA,
        ],
        compiler_params=pltpu.CompilerParams(
            needs_layout_passes=False, use_tc_tiling_on_sc=True),
    )
    def gather_kernel(ptab_hbm, idx_hbm, out_hbm, ptab_v,
                      idx_v0, idx_v1, out_v0, out_v1,
                      sem_i0, sem_i1, sem_o0, sem_o1):
        wid = lax.axis_index("s") * _NC + lax.axis_index("c")
        base = wid * rows_per_w
        idx_bufs, out_bufs = (idx_v0, idx_v1), (out_v0, out_v1)
        sem_in, sem_out = (sem_i0, sem_i1), (sem_o0, sem_o1)
        pltpu.sync_copy(ptab_hbm, ptab_v)

        def in_copy(g):
            return pltpu.async_copy(
                idx_hbm.at[pl.ds(base + g * _RCHUNK, _RCHUNK)],
                idx_bufs[g % 2], sem_in[g % 2])

        in_handles = [in_copy(0)]
        out_handles = [None] * nchunk
        for g in range(nchunk):
            cur = g % 2
            if g + 1 < nchunk:
                in_handles.append(in_copy(g + 1))
            in_handles[g].wait()
            if g >= 2:
                out_handles[g - 2].wait()
            idx_v, out_v = idx_bufs[cur], out_bufs[cur]

            def inner(r, c, idx_v=idx_v, out_v=out_v):
                for s in starts:
                    sl = pl.ds(s, _LANES)
                    out_v[r, sl] = plsc.load_gather(ptab_v, [idx_v[r, sl]])
                return c

            lax.fori_loop(0, _RCHUNK, inner, 0)
            out_handles[g] = pltpu.async_copy(
                out_v, out_hbm.at[pl.ds(base + g * _RCHUNK, _RCHUNK)],
                sem_out[cur])
        for g in range(max(0, nchunk - 2), nchunk):
            out_handles[g].wait()

    return gather_kernel


def kernel(pos_indices, emb_table, W, b):
    batch, seqlen = pos_indices.shape
    idx = pos_indices.astype(jnp.int32)
    ptab = _compute_prob_table(emb_table, W, b).reshape(-1)
    out = _make_sc_gather(batch, seqlen, emb_table.shape[0])(ptab, idx)
    return out[..., None]


# speedup vs baseline: 192.0287x; 1.2515x over previous
import functools

import jax
import jax.numpy as jnp
from jax import lax
from jax.experimental import pallas as pl
from jax.experimental.pallas import tpu as pltpu
from jax.experimental.pallas import tpu_sc as plsc

_NC = 2
_NS = 16
_NW = _NC * _NS
_LANES = 16
_RCHUNK = 64
_NBUF = 3


def _prob_body(tab_ref, w_ref, b_ref, out_ref):
    t = tab_ref[...]
    w = w_ref[...]
    logits = jnp.sum(t * w, axis=1, keepdims=True) + b_ref[...]
    out_ref[...] = jax.nn.sigmoid(logits)


def _compute_prob_table(emb_table, W, b):
    rows = emb_table.shape[0]
    return pl.pallas_call(
        _prob_body,
        out_shape=jax.ShapeDtypeStruct((rows, 1), jnp.float32),
    )(emb_table, W, b.reshape(1, 1))


@functools.lru_cache(maxsize=None)
def _make_sc_gather(batch, seqlen, table_rows):
    assert batch % (_NW * _RCHUNK) == 0
    rows_per_w = batch // _NW
    nchunk = rows_per_w // _RCHUNK
    starts = list(range(0, (seqlen // _LANES) * _LANES, _LANES))
    if seqlen % _LANES:
        starts.append(seqlen - _LANES)
    mesh = plsc.VectorSubcoreMesh(core_axis_name="c", subcore_axis_name="s")

    @functools.partial(
        pl.kernel,
        out_type=jax.ShapeDtypeStruct((batch, seqlen), jnp.float32),
        mesh=mesh,
        scratch_types=(
            [pltpu.VMEM((table_rows,), jnp.float32)]
            + [pltpu.VMEM((_RCHUNK, seqlen), jnp.int32) for _ in range(_NBUF)]
            + [pltpu.VMEM((_RCHUNK, seqlen), jnp.float32) for _ in range(_NBUF)]
            + [pltpu.SemaphoreType.DMA for _ in range(2 * _NBUF)]
        ),
        compiler_params=pltpu.CompilerParams(
            needs_layout_passes=False, use_tc_tiling_on_sc=True),
    )
    def gather_kernel(ptab_hbm, idx_hbm, out_hbm, ptab_v, *scratch):
        idx_bufs = scratch[:_NBUF]
        out_bufs = scratch[_NBUF:2 * _NBUF]
        sem_in = scratch[2 * _NBUF:3 * _NBUF]
        sem_out = scratch[3 * _NBUF:4 * _NBUF]
        wid = lax.axis_index("s") * _NC + lax.axis_index("c")
        base = wid * rows_per_w
        pltpu.sync_copy(ptab_hbm, ptab_v)

        def in_copy(g):
            return pltpu.async_copy(
                idx_hbm.at[pl.ds(base + g * _RCHUNK, _RCHUNK)],
                idx_bufs[g % _NBUF], sem_in[g % _NBUF])

        in_handles = [in_copy(g) for g in range(min(_NBUF, nchunk))]
        out_handles = [None] * nchunk
        for g in range(nchunk):
            cur = g % _NBUF
            in_handles[g].wait()
            if g >= _NBUF:
                out_handles[g - _NBUF].wait()
            idx_v, out_v = idx_bufs[cur], out_bufs[cur]

            def inner(r, c, idx_v=idx_v, out_v=out_v):
                ivs = [idx_v[r, pl.ds(s, _LANES)] for s in starts]
                gs = [plsc.load_gather(ptab_v, [iv]) for iv in ivs]
                for s, g_ in zip(starts, gs):
                    out_v[r, pl.ds(s, _LANES)] = g_
                return c

            lax.fori_loop(0, _RCHUNK, inner, 0)
            out_handles[g] = pltpu.async_copy(
                out_v, out_hbm.at[pl.ds(base + g * _RCHUNK, _RCHUNK)],
                sem_out[cur])
            if g + _NBUF < nchunk:
                in_handles.append(in_copy(g + _NBUF))
        for g in range(max(0, nchunk - _NBUF), nchunk):
            out_handles[g].wait()

    return gather_kernel


def kernel(pos_indices, emb_table, W, b):
    batch, seqlen = pos_indices.shape
    idx = pos_indices.astype(jnp.int32)
    ptab = _compute_prob_table(emb_table, W, b).reshape(-1)
    out = _make_sc_gather(batch, seqlen, emb_table.shape[0])(ptab, idx)
    return out[..., None]
